# Initial kernel scaffold; baseline (speedup 1.0000x reference)
import jax
import jax.numpy as jnp
from jax.experimental import pallas as pl


def kernel(x, cell_id, coordinates, nodal_values, connectivity):
    raise NotImplementedError("write your pallas kernel here")


# speedup vs baseline: 10.3361x; 10.3361x over previous
import functools

import jax
import jax.numpy as jnp
from jax import lax
from jax.experimental import pallas as pl
from jax.experimental.pallas import tpu as pltpu
from jax.experimental.pallas import tpu_sc as plsc

N_ELEM = 64
N_NODES = 66
L = 16
NC = 2
NS = 16
NW = NC * NS


def _body(x_hbm, cid_hbm, coord_hbm, conn_hbm, out_hbm,
          coord_v, conn_v, tbl_v, x_v, cid_v, out_v, n_per_w):
    wid = lax.axis_index("s") * NC + lax.axis_index("c")
    base = wid * n_per_w

    pltpu.sync_copy(coord_hbm, coord_v)
    pltpu.sync_copy(conn_hbm, conn_v)
    pltpu.sync_copy(x_hbm.at[pl.ds(base * 2, n_per_w * 2)], x_v)
    pltpu.sync_copy(cid_hbm.at[pl.ds(base, n_per_w)], cid_v)

    for j in range(N_ELEM // L):
        e = lax.iota(jnp.int32, L) + (j * L)
        c1 = plsc.load_gather(conn_v, [e * 3]) - 1
        c2 = plsc.load_gather(conn_v, [e * 3 + 1]) - 1
        c3 = plsc.load_gather(conn_v, [e * 3 + 2]) - 1
        x1 = plsc.load_gather(coord_v, [c1 * 2])
        y1 = plsc.load_gather(coord_v, [c1 * 2 + 1])
        x2 = plsc.load_gather(coord_v, [c2 * 2])
        y2 = plsc.load_gather(coord_v, [c2 * 2 + 1])
        x3 = plsc.load_gather(coord_v, [c3 * 2])
        y3 = plsc.load_gather(coord_v, [c3 * 2 + 1])
        d1 = x1 * (y3 - y2) + x2 * (y1 - y3) + x3 * (y2 - y1)
        d2 = -x1 * y2 + x1 * y3 + x2 * y1 - x2 * y3 - x3 * y1 + x3 * y2
        d3 = x1 * (y2 - y3) + x2 * (y3 - y1) + x3 * (y1 - y2)
        ms = (
            (y3 - y2) / d1, (x2 - x3) / d2, (x3 * y2 - x2 * y3) / d2,
            (y1 - y3) / d2, (x1 - x3) / d3, (x3 * y1 - x1 * y3) / d3,
            (y1 - y2) / d3, (x1 - x2) / d2, (x2 * y1 - x1 * y2) / d2,
        )
        for k, m in enumerate(ms):
            tbl_v[pl.ds(k * N_ELEM + j * L, L)] = m

    def step(i, _):
        off = i * L
        lane = lax.iota(jnp.int32, L) + off
        cid = cid_v[pl.ds(off, L)]
        px = plsc.load_gather(x_v, [lane * 2])
        py = plsc.load_gather(x_v, [lane * 2 + 1])
        t0 = plsc.load_gather(tbl_v, [cid])
        t1 = plsc.load_gather(tbl_v, [cid + N_ELEM])
        t2 = plsc.load_gather(tbl_v, [cid + 2 * N_ELEM])
        t3 = plsc.load_gather(tbl_v, [cid + 3 * N_ELEM])
        t4 = plsc.load_gather(tbl_v, [cid + 4 * N_ELEM])
        t5 = plsc.load_gather(tbl_v, [cid + 5 * N_ELEM])
        t6 = plsc.load_gather(tbl_v, [cid + 6 * N_ELEM])
        t7 = plsc.load_gather(tbl_v, [cid + 7 * N_ELEM])
        t8 = plsc.load_gather(tbl_v, [cid + 8 * N_ELEM])
        r0 = px * t0 + py * t1 + t2
        r1 = px * t3 + py * t4 + t5
        r2 = px * t6 + py * t7 + t8
        o = lane * 3
        plsc.store_scatter(out_v, [o], r0)
        plsc.store_scatter(out_v, [o + 1], r1)
        plsc.store_scatter(out_v, [o + 2], r2)
        return _

    lax.fori_loop(0, n_per_w // L, step, None)

    pltpu.sync_copy(out_v, out_hbm.at[pl.ds(base * 3, n_per_w * 3)])


def kernel(x, cell_id, coordinates, nodal_values, connectivity):
    del nodal_values
    B = x.shape[0]
    n_per_w = B // NW

    coord_flat = coordinates.reshape(-1)
    conn_flat = connectivity.reshape(-1)
    x_flat = x.reshape(-1)

    mesh = plsc.VectorSubcoreMesh(core_axis_name="c", subcore_axis_name="s")
    run = functools.partial(
        pl.kernel,
        out_type=jax.ShapeDtypeStruct((B * 3,), jnp.float32),
        mesh=mesh,
        compiler_params=pltpu.CompilerParams(needs_layout_passes=False),
        scratch_types=[
            pltpu.VMEM((coord_flat.shape[0],), jnp.float32),
            pltpu.VMEM((conn_flat.shape[0],), jnp.int32),
            pltpu.VMEM((9 * N_ELEM,), jnp.float32),
            pltpu.VMEM((n_per_w * 2,), jnp.float32),
            pltpu.VMEM((n_per_w,), jnp.int32),
            pltpu.VMEM((n_per_w * 3,), jnp.float32),
        ],
    )(functools.partial(_body, n_per_w=n_per_w))
    out = run(x_flat, cell_id, coord_flat, conn_flat)
    return out.reshape(B, 3)


# speedup vs baseline: 99.4260x; 9.6193x over previous
import functools

import jax
import jax.numpy as jnp
from jax import lax
from jax.experimental import pallas as pl
from jax.experimental.pallas import tpu as pltpu
from jax.experimental.pallas import tpu_sc as plsc

N_ELEM = 64
N_NODES = 66
L = 16
NC = 2
NS = 16
NW = NC * NS
BLK = 128


def _body(x_hbm, cid_hbm, coord_hbm, conn_hbm, out_hbm,
          coord_v, conn_v, tbl_v, x_v, cid_v, out_v, blk_per_w):
    wid = lax.axis_index("s") * NC + lax.axis_index("c")
    n_per_w = blk_per_w * BLK

    pltpu.sync_copy(coord_hbm, coord_v)
    pltpu.sync_copy(conn_hbm, conn_v)
    pltpu.sync_copy(x_hbm.at[pl.ds(wid * (n_per_w * 2), n_per_w * 2)], x_v)
    pltpu.sync_copy(cid_hbm.at[pl.ds(wid * n_per_w, n_per_w)], cid_v)

    for j in range(N_ELEM // L):
        e = lax.iota(jnp.int32, L) + (j * L)
        c1 = plsc.load_gather(conn_v, [e * 3]) - 1
        c2 = plsc.load_gather(conn_v, [e * 3 + 1]) - 1
        c3 = plsc.load_gather(conn_v, [e * 3 + 2]) - 1
        x1 = plsc.load_gather(coord_v, [c1 * 2])
        y1 = plsc.load_gather(coord_v, [c1 * 2 + 1])
        x2 = plsc.load_gather(coord_v, [c2 * 2])
        y2 = plsc.load_gather(coord_v, [c2 * 2 + 1])
        x3 = plsc.load_gather(coord_v, [c3 * 2])
        y3 = plsc.load_gather(coord_v, [c3 * 2 + 1])
        d1 = x1 * (y3 - y2) + x2 * (y1 - y3) + x3 * (y2 - y1)
        d2 = -x1 * y2 + x1 * y3 + x2 * y1 - x2 * y3 - x3 * y1 + x3 * y2
        d3 = x1 * (y2 - y3) + x2 * (y3 - y1) + x3 * (y1 - y2)
        ms = (
            (y3 - y2) / d1, (x2 - x3) / d2, (x3 * y2 - x2 * y3) / d2,
            (y1 - y3) / d2, (x1 - x3) / d3, (x3 * y1 - x1 * y3) / d3,
            (y1 - y2) / d3, (x1 - x2) / d2, (x2 * y1 - x1 * y2) / d2,
        )
        for k, m in enumerate(ms):
            tbl_v[pl.ds(k * N_ELEM + j * L, L)] = m

    def step(b, _):
        xo = b * (2 * BLK)
        oo = b * (4 * BLK)
        co = b * BLK
        for s in range(BLK // L):
            px = x_v[pl.ds(xo + s * L, L)]
            py = x_v[pl.ds(xo + BLK + s * L, L)]
            cid = cid_v[pl.ds(co + s * L, L)]
            t0 = plsc.load_gather(tbl_v, [cid])
            t1 = plsc.load_gather(tbl_v, [cid + N_ELEM])
            t2 = plsc.load_gather(tbl_v, [cid + 2 * N_ELEM])
            t3 = plsc.load_gather(tbl_v, [cid + 3 * N_ELEM])
            t4 = plsc.load_gather(tbl_v, [cid + 4 * N_ELEM])
            t5 = plsc.load_gather(tbl_v, [cid + 5 * N_ELEM])
            t6 = plsc.load_gather(tbl_v, [cid + 6 * N_ELEM])
            t7 = plsc.load_gather(tbl_v, [cid + 7 * N_ELEM])
            t8 = plsc.load_gather(tbl_v, [cid + 8 * N_ELEM])
            out_v[pl.ds(oo + s * L, L)] = px * t0 + py * t1 + t2
            out_v[pl.ds(oo + BLK + s * L, L)] = px * t3 + py * t4 + t5
            out_v[pl.ds(oo + 2 * BLK + s * L, L)] = px * t6 + py * t7 + t8
        return _

    lax.fori_loop(0, blk_per_w, step, None)

    pltpu.sync_copy(out_v, out_hbm.at[pl.ds(wid * (n_per_w * 4), n_per_w * 4)])


def kernel(x, cell_id, coordinates, nodal_values, connectivity):
    del nodal_values
    B = x.shape[0]
    nblk = B // BLK
    blk_per_w = nblk // NW

    x_raw = x.reshape(nblk, BLK, 2).transpose(0, 2, 1).reshape(-1)
    coord_flat = coordinates.reshape(-1)
    conn_flat = connectivity.reshape(-1)

    mesh = plsc.VectorSubcoreMesh(core_axis_name="c", subcore_axis_name="s")
    run = functools.partial(
        pl.kernel,
        out_type=jax.ShapeDtypeStruct((B * 4,), jnp.float32),
        mesh=mesh,
        compiler_params=pltpu.CompilerParams(needs_layout_passes=False),
        scratch_types=[
            pltpu.VMEM((coord_flat.shape[0],), jnp.float32),
            pltpu.VMEM((conn_flat.shape[0],), jnp.int32),
            pltpu.VMEM((9 * N_ELEM,), jnp.float32),
            pltpu.VMEM((blk_per_w * 2 * BLK,), jnp.float32),
            pltpu.VMEM((blk_per_w * BLK,), jnp.int32),
            pltpu.VMEM((blk_per_w * 4 * BLK,), jnp.float32),
        ],
    )(functools.partial(_body, blk_per_w=blk_per_w))
    out_raw = run(x_raw, cell_id, coord_flat, conn_flat)
    out = out_raw.reshape(nblk, 4, BLK).transpose(0, 2, 1)[:, :, :3]
    return out.reshape(B, 3)


# speedup vs baseline: 124.2572x; 1.2497x over previous
import functools

import jax
import jax.numpy as jnp
from jax import lax
from jax.experimental import pallas as pl
from jax.experimental.pallas import tpu as pltpu
from jax.experimental.pallas import tpu_sc as plsc

N_ELEM = 64
N_NODES = 66
L = 16
NC = 2
NS = 16
NW = NC * NS
BLK = 128


def _body(x_hbm, cid_hbm, coord_hbm, conn_hbm, out_hbm,
          coord_v, conn_v, tbl_v, x_v, cid_v, out_v, sem_x, sem_c, blk_per_w):
    wid = lax.axis_index("s") * NC + lax.axis_index("c")
    n_per_w = blk_per_w * BLK

    cp_x = pltpu.async_copy(
        x_hbm.at[pl.ds(wid * (n_per_w * 2), n_per_w * 2)], x_v, sem_x)
    cp_c = pltpu.async_copy(
        cid_hbm.at[pl.ds(wid * n_per_w, n_per_w)], cid_v, sem_c)
    pltpu.sync_copy(coord_hbm, coord_v)
    pltpu.sync_copy(conn_hbm, conn_v)

    for j in range(N_ELEM // L):
        e = lax.iota(jnp.int32, L) + (j * L)
        c1 = plsc.load_gather(conn_v, [e * 3]) - 1
        c2 = plsc.load_gather(conn_v, [e * 3 + 1]) - 1
        c3 = plsc.load_gather(conn_v, [e * 3 + 2]) - 1
        x1 = plsc.load_gather(coord_v, [c1 * 2])
        y1 = plsc.load_gather(coord_v, [c1 * 2 + 1])
        x2 = plsc.load_gather(coord_v, [c2 * 2])
        y2 = plsc.load_gather(coord_v, [c2 * 2 + 1])
        x3 = plsc.load_gather(coord_v, [c3 * 2])
        y3 = plsc.load_gather(coord_v, [c3 * 2 + 1])
        d1 = x1 * (y3 - y2) + x2 * (y1 - y3) + x3 * (y2 - y1)
        d2 = -x1 * y2 + x1 * y3 + x2 * y1 - x2 * y3 - x3 * y1 + x3 * y2
        d3 = x1 * (y2 - y3) + x2 * (y3 - y1) + x3 * (y1 - y2)
        ms = (
            (y3 - y2) / d1, (x2 - x3) / d2, (x3 * y2 - x2 * y3) / d2,
            (y1 - y3) / d2, (x1 - x3) / d3, (x3 * y1 - x1 * y3) / d3,
        )
        for k, m in enumerate(ms):
            tbl_v[pl.ds(k * N_ELEM + j * L, L)] = m

    cp_x.wait()
    cp_c.wait()

    @plsc.parallel_loop(0, blk_per_w, 1)
    def step(b):
        xo = b * (2 * BLK)
        oo = b * (4 * BLK)
        co = b * BLK
        for s in range(BLK // L):
            px = x_v[pl.ds(xo + s * L, L)]
            py = x_v[pl.ds(xo + BLK + s * L, L)]
            cid = cid_v[pl.ds(co + s * L, L)]
            t0 = plsc.load_gather(tbl_v, [cid])
            t1 = plsc.load_gather(tbl_v, [cid + N_ELEM])
            t2 = plsc.load_gather(tbl_v, [cid + 2 * N_ELEM])
            t3 = plsc.load_gather(tbl_v, [cid + 3 * N_ELEM])
            t4 = plsc.load_gather(tbl_v, [cid + 4 * N_ELEM])
            t5 = plsc.load_gather(tbl_v, [cid + 5 * N_ELEM])
            r0 = px * t0 + py * t1 + t2
            r1 = px * t3 + py * t4 + t5
            out_v[pl.ds(oo + s * L, L)] = r0
            out_v[pl.ds(oo + BLK + s * L, L)] = r1
            out_v[pl.ds(oo + 2 * BLK + s * L, L)] = 1.0 - r0 - r1

    pltpu.sync_copy(out_v, out_hbm.at[pl.ds(wid * (n_per_w * 4), n_per_w * 4)])


def kernel(x, cell_id, coordinates, nodal_values, connectivity):
    del nodal_values
    B = x.shape[0]
    nblk = B // BLK
    blk_per_w = nblk // NW

    x_raw = x.reshape(nblk, BLK, 2).transpose(0, 2, 1).reshape(-1)
    coord_flat = coordinates.reshape(-1)
    conn_flat = connectivity.reshape(-1)

    mesh = plsc.VectorSubcoreMesh(core_axis_name="c", subcore_axis_name="s")
    run = functools.partial(
        pl.kernel,
        out_type=jax.ShapeDtypeStruct((B * 4,), jnp.float32),
        mesh=mesh,
        compiler_params=pltpu.CompilerParams(needs_layout_passes=False),
        scratch_types=[
            pltpu.VMEM((coord_flat.shape[0],), jnp.float32),
            pltpu.VMEM((conn_flat.shape[0],), jnp.int32),
            pltpu.VMEM((6 * N_ELEM,), jnp.float32),
            pltpu.VMEM((blk_per_w * 2 * BLK,), jnp.float32),
            pltpu.VMEM((blk_per_w * BLK,), jnp.int32),
            pltpu.VMEM((blk_per_w * 4 * BLK,), jnp.float32),
            pltpu.SemaphoreType.DMA,
            pltpu.SemaphoreType.DMA,
        ],
    )(functools.partial(_body, blk_per_w=blk_per_w))
    out_raw = run(x_raw, cell_id, coord_flat, conn_flat)
    out = out_raw.reshape(nblk, 4, BLK).transpose(0, 2, 1)[:, :, :3]
    return out.reshape(B, 3)
